# baseline (speedup 1.0000x reference)
import functools

import jax
import jax.numpy as jnp
from jax import lax
from jax.experimental import pallas as pl
from jax.experimental.pallas import tpu as pltpu
from jax.experimental.pallas import tpu_sc as plsc

N = 10000
F = 128
H = 256
C = 64
E = 320000

NC = 2
NS = 16
NW = NC * NS
EB = 128
NB2 = 80
NB1 = 160
E_PAD = NW * NB2 * EB
N_ACC = 10112
RPT = N_ACC // NS
HD = 64

_mesh = plsc.VectorSubcoreMesh(core_axis_name="c", subcore_axis_name="s")
_sc_params = pltpu.CompilerParams(needs_layout_passes=False,
                                  use_tc_tiling_on_sc=False)


@functools.partial(
    pl.kernel,
    out_type=jax.ShapeDtypeStruct((NW, N_ACC), jnp.float32),
    mesh=_mesh,
    compiler_params=_sc_params,
    scratch_types=[
        pltpu.VMEM((NB2 * EB,), jnp.int32),
        pltpu.VMEM((N_ACC,), jnp.float32),
    ],
)
def _deg_kernel(dst_hbm, out_hbm, dst_v, deg_v):
    c = lax.axis_index("c")
    s = lax.axis_index("s")
    wid = c * NS + s
    pltpu.sync_copy(dst_hbm.at[wid], dst_v)

    zero16 = jnp.zeros((16,), jnp.float32)

    def zbody(i, _):
        deg_v[pl.ds(i * 16, 16)] = zero16
        return 0

    lax.fori_loop(0, N_ACC // 16, zbody, 0)

    one16 = jnp.ones((16,), jnp.float32)

    def body(i, _):
        idx = dst_v[pl.ds(i * 16, 16)]
        plsc.addupdate_scatter(deg_v, [idx], one16)
        return 0

    lax.fori_loop(0, (NB2 * EB) // 16, body, 0)
    pltpu.sync_copy(deg_v, out_hbm.at[wid])


def _make_agg_kernel(nb, feature_split):
    tshape = (NC, N, HD) if feature_split else (N, HD)

    @functools.partial(
        pl.kernel,
        out_type=jax.ShapeDtypeStruct((NC, N_ACC, HD), jnp.float32),
        mesh=_mesh,
        compiler_params=_sc_params,
        scratch_types=[
            pltpu.VMEM((nb, EB), jnp.int32),
            pltpu.VMEM((nb, EB), jnp.int32),
        ] + [pltpu.VMEM((EB, HD), jnp.float32)] * 4
          + [pltpu.VMEM_SHARED((N_ACC, HD), jnp.float32)]
          + [pltpu.SemaphoreType.DMA] * 8,
    )
    def _agg(table_hbm, src_hbm, dst_hbm, zeros_hbm, out_hbm,
             src_v, dst_v, buf0, buf1, buf2, buf3, acc_sh,
             sg0, sg1, sg2, sg3, ss0, ss1, ss2, ss3):
        bufs = (buf0, buf1, buf2, buf3)
        sgs = (sg0, sg1, sg2, sg3)
        sss = (ss0, ss1, ss2, ss3)
        c = lax.axis_index("c")
        s = lax.axis_index("s")
        slab = s if feature_split else c * NS + s
        table = table_hbm.at[c] if feature_split else table_hbm
        pltpu.sync_copy(src_hbm.at[slab], src_v)
        pltpu.sync_copy(dst_hbm.at[slab], dst_v)
        pltpu.sync_copy(zeros_hbm.at[pl.ds(s * RPT, RPT)],
                        acc_sh.at[pl.ds(s * RPT, RPT)])
        plsc.subcore_barrier()

        for j in range(4):
            pltpu.async_copy(table.at[src_v.at[j]], bufs[j], sgs[j])

        def body(i, _):
            base = 4 * i
            for j in range(4):
                b = base + j
                pltpu.make_async_copy(
                    table.at[src_v.at[b]], bufs[j], sgs[j]).wait()
                pltpu.async_copy(
                    bufs[j], acc_sh.at[dst_v.at[b]], sss[j], add=True)
            for j in range(4):
                b = base + j
                pltpu.make_async_copy(
                    bufs[j], acc_sh.at[dst_v.at[b]], sss[j]).wait()

                def _start_next(j=j, b=b):
                    pltpu.async_copy(
                        table.at[src_v.at[b + 4]], bufs[j], sgs[j])

                pl.when(b + 4 < nb)(_start_next)
            return 0

        lax.fori_loop(0, nb // 4, body, 0)
        plsc.subcore_barrier()
        pltpu.sync_copy(acc_sh.at[pl.ds(s * RPT, RPT)],
                        out_hbm.at[c, pl.ds(s * RPT, RPT)])

    return _agg


_agg1 = _make_agg_kernel(NB1, True)
_agg2 = _make_agg_kernel(NB2, False)


def _dinv_from_parts(degp):
    deg = jnp.sum(degp, axis=0)[:N] + 1.0
    return lax.rsqrt(deg)[:, None]


def _tc1_body(degp_ref, x_ref, y1_ref):
    dinv = _dinv_from_parts(degp_ref[...])
    y1_ref[0] = x_ref[:, :HD] * dinv
    y1_ref[1] = x_ref[:, HD:] * dinv


def _tc1(deg_parts, x):
    return pl.pallas_call(
        _tc1_body,
        out_shape=jax.ShapeDtypeStruct((NC, N, HD), jnp.float32),
    )(deg_parts, x)


def _tc2_body(acc_ref, y1_ref, degp_ref, w1_ref, b1_ref, w2_ref, b2_ref, y2_ref):
    dinv = _dinv_from_parts(degp_ref[...])
    z = jnp.concatenate(
        [acc_ref[0, :N, :] + y1_ref[0], acc_ref[1, :N, :] + y1_ref[1]],
        axis=1) * dinv
    h = jnp.dot(z, w1_ref[...], preferred_element_type=jnp.float32) + b1_ref[...]
    h = jnp.maximum(h, 0.0)
    y2 = jnp.dot(h, w2_ref[...], preferred_element_type=jnp.float32)
    y2_ref[...] = y2 * dinv


def _tc2(acc1, y1_two, deg_parts, W1, b1, W2, b2):
    return pl.pallas_call(
        _tc2_body,
        out_shape=jax.ShapeDtypeStruct((N, C), jnp.float32),
    )(acc1, y1_two, deg_parts, W1, b1, W2, b2)


def _tc3_body(acc_ref, y2_ref, degp_ref, b2_ref, out_ref):
    dinv = _dinv_from_parts(degp_ref[...])
    out_ref[...] = (acc_ref[0, :N, :] + acc_ref[1, :N, :]
                    + y2_ref[...]) * dinv + b2_ref[...]


def _tc3(acc2, y2, deg_parts, b2):
    return pl.pallas_call(
        _tc3_body,
        out_shape=jax.ShapeDtypeStruct((N, C), jnp.float32),
    )(acc2, y2, deg_parts, b2)


def kernel(x, edge_index, W1, b1, W2, b2):
    ei = edge_index.astype(jnp.int32)
    src, dst = ei[0], ei[1]
    pad = E_PAD - E
    src_pad = jnp.concatenate([src, jnp.zeros((pad,), jnp.int32)])
    dst_pad = jnp.concatenate([dst, jnp.full((pad,), N, jnp.int32)])
    src32 = src_pad.reshape(NW, NB2, EB)
    dst32 = dst_pad.reshape(NW, NB2, EB)
    src16 = src_pad.reshape(NS, NB1, EB)
    dst16 = dst_pad.reshape(NS, NB1, EB)
    dst_flat = dst_pad.reshape(NW, NB2 * EB)

    zeros_hd = jnp.zeros((N_ACC, HD), jnp.float32)

    deg_parts = _deg_kernel(dst_flat)
    y1_two = _tc1(deg_parts, x)
    acc1 = _agg1(y1_two, src16, dst16, zeros_hd)
    y2 = _tc2(acc1, y1_two, deg_parts,
              W1, b1.reshape(1, H), W2, b2.reshape(1, C))
    acc2 = _agg2(y2, src32, dst32, zeros_hd)
    return _tc3(acc2, y2, deg_parts, b2.reshape(1, C))


# speedup vs baseline: 1.5603x; 1.5603x over previous
import functools

import jax
import jax.numpy as jnp
from jax import lax
from jax.experimental import pallas as pl
from jax.experimental.pallas import tpu as pltpu
from jax.experimental.pallas import tpu_sc as plsc

N = 10000
F = 128
H = 256
C = 64
E = 320000

NC = 2
NS = 16
NW = NC * NS
EB = 128
NB = 160
E_PAD = NS * NB * EB
N_ACC = 10112
RPT = N_ACC // NS
SRT = N // NS
GW = 32

_mesh = plsc.VectorSubcoreMesh(core_axis_name="c", subcore_axis_name="s")
_sc_params = pltpu.CompilerParams(needs_layout_passes=False,
                                  use_tc_tiling_on_sc=False)


@functools.partial(
    pl.kernel,
    out_type=jax.ShapeDtypeStruct((NW, N_ACC), jnp.float32),
    mesh=_mesh,
    compiler_params=_sc_params,
    scratch_types=[
        pltpu.VMEM((NB * EB // 2,), jnp.int32),
        pltpu.VMEM((N_ACC,), jnp.float32),
    ],
)
def _deg_kernel(dst_hbm, out_hbm, dst_v, deg_v):
    c = lax.axis_index("c")
    s = lax.axis_index("s")
    wid = c * NS + s
    pltpu.sync_copy(dst_hbm.at[wid], dst_v)

    zero16 = jnp.zeros((16,), jnp.float32)

    def zbody(i, _):
        deg_v[pl.ds(i * 16, 16)] = zero16
        return 0

    lax.fori_loop(0, N_ACC // 16, zbody, 0)

    one16 = jnp.ones((16,), jnp.float32)

    def body(i, _):
        idx = dst_v[pl.ds(i * 16, 16)]
        plsc.addupdate_scatter(deg_v, [idx], one16)
        return 0

    lax.fori_loop(0, (NB * EB // 2) // 16, body, 0)
    pltpu.sync_copy(deg_v, out_hbm.at[wid])


def _make_agg_kernel(n_groups, scale):
    ng_total = NC * n_groups
    extra_in = [None] if scale else []

    @functools.partial(
        pl.kernel,
        out_type=jax.ShapeDtypeStruct((ng_total, N_ACC, GW), jnp.float32),
        mesh=_mesh,
        compiler_params=_sc_params,
        scratch_types=[
            pltpu.VMEM((NB, EB), jnp.int32),
            pltpu.VMEM((NB, EB), jnp.int32),
            pltpu.VMEM((SRT, GW), jnp.float32),
            pltpu.VMEM((640,), jnp.float32),
        ] + [pltpu.VMEM((EB, GW), jnp.float32)] * 4
          + [pltpu.VMEM_SHARED((N, GW), jnp.float32)]
          + [pltpu.VMEM_SHARED((N_ACC, GW), jnp.float32)]
          + [pltpu.SemaphoreType.DMA] * 8,
    )
    def _agg(table_hbm, *args):
        if scale:
            dinv_hbm, src_hbm, dst_hbm, zeros_hbm, out_hbm = args[:5]
        else:
            src_hbm, dst_hbm, zeros_hbm, out_hbm = args[:4]
            dinv_hbm = None
        rest = args[5:] if scale else args[4:]
        src_v, dst_v, stage_v, dinv_v = rest[:4]
        bufs = rest[4:8]
        table = rest[8]
        acc_sh = rest[9]
        sgs = rest[10:14]
        sss = rest[14:18]

        c = lax.axis_index("c")
        s = lax.axis_index("s")
        pltpu.sync_copy(src_hbm.at[s], src_v)
        pltpu.sync_copy(dst_hbm.at[s], dst_v)
        if scale:
            pltpu.sync_copy(dinv_hbm.at[s], dinv_v)

        for g in range(n_groups):
            src_rows = table_hbm.at[c * n_groups + g, pl.ds(s * SRT, SRT)]
            if scale:
                pltpu.sync_copy(src_rows, stage_v)

                def srow(r, _):
                    base = pl.multiple_of((r // 16) * 16, 16)
                    dvec = dinv_v[pl.ds(base, 16)]
                    lane = jnp.full((16,), r - base, jnp.int32)
                    d = dvec.at[lane].get(mode="promise_in_bounds")
                    stage_v[r, pl.ds(0, 16)] = stage_v[r, pl.ds(0, 16)] * d
                    stage_v[r, pl.ds(16, 16)] = stage_v[r, pl.ds(16, 16)] * d
                    return 0

                lax.fori_loop(0, SRT, srow, 0)
                pltpu.sync_copy(stage_v, table.at[pl.ds(s * SRT, SRT)])
            else:
                pltpu.sync_copy(src_rows, table.at[pl.ds(s * SRT, SRT)])
            pltpu.sync_copy(zeros_hbm.at[pl.ds(s * RPT, RPT)],
                            acc_sh.at[pl.ds(s * RPT, RPT)])
            plsc.subcore_barrier()

            for j in range(4):
                pltpu.async_copy(table.at[src_v.at[j]], bufs[j], sgs[j])

            def body(i, _):
                base = 4 * i
                for j in range(4):
                    b = base + j
                    pltpu.make_async_copy(
                        table.at[src_v.at[b]], bufs[j], sgs[j]).wait()
                    pltpu.async_copy(
                        bufs[j], acc_sh.at[dst_v.at[b]], sss[j], add=True)
                for j in range(4):
                    b = base + j
                    pltpu.make_async_copy(
                        bufs[j], acc_sh.at[dst_v.at[b]], sss[j]).wait()

                    def _start_next(j=j, b=b):
                        pltpu.async_copy(
                            table.at[src_v.at[b + 4]], bufs[j], sgs[j])

                    pl.when(b + 4 < NB)(_start_next)
                return 0

            lax.fori_loop(0, NB // 4, body, 0)
            plsc.subcore_barrier()
            pltpu.sync_copy(acc_sh.at[pl.ds(s * RPT, RPT)],
                            out_hbm.at[c * n_groups + g, pl.ds(s * RPT, RPT)])

    return _agg


_agg1 = _make_agg_kernel(2, True)
_agg2 = _make_agg_kernel(1, False)


def _dinv_from_parts(degp):
    deg = jnp.sum(degp, axis=0)[:N] + 1.0
    return lax.rsqrt(deg)[:, None]


def _tc_dinv_body(degp_ref, dinv_ref):
    deg = jnp.sum(degp_ref[...], axis=0) + 1.0
    dinv_ref[...] = lax.rsqrt(deg).reshape(N_ACC // 128, 128)


def _tc_dinv(deg_parts):
    return pl.pallas_call(
        _tc_dinv_body,
        out_shape=jax.ShapeDtypeStruct((N_ACC // 128, 128), jnp.float32),
    )(deg_parts)


def _tc2_body(acc_ref, x_ref, degp_ref, w1_ref, b1_ref, w2_ref, b2_ref, y2_ref):
    dinv = _dinv_from_parts(degp_ref[...])
    agg = jnp.concatenate([acc_ref[g, :N, :] for g in range(4)], axis=1)
    z = (agg + x_ref[...] * dinv) * dinv
    h = jnp.dot(z, w1_ref[...], preferred_element_type=jnp.float32) + b1_ref[...]
    h = jnp.maximum(h, 0.0)
    y2 = jnp.dot(h, w2_ref[...], preferred_element_type=jnp.float32) * dinv
    y2_ref[0] = y2[:, :GW]
    y2_ref[1] = y2[:, GW:]


def _tc2(acc1, x, deg_parts, W1, b1, W2, b2):
    return pl.pallas_call(
        _tc2_body,
        out_shape=jax.ShapeDtypeStruct((NC, N, GW), jnp.float32),
    )(acc1, x, deg_parts, W1, b1, W2, b2)


def _tc3_body(acc_ref, y2_ref, degp_ref, b2_ref, out_ref):
    dinv = _dinv_from_parts(degp_ref[...])
    s = jnp.concatenate(
        [acc_ref[g, :N, :] + y2_ref[g] for g in range(NC)], axis=1)
    out_ref[...] = s * dinv + b2_ref[...]


def _tc3(acc2, y2g, deg_parts, b2):
    return pl.pallas_call(
        _tc3_body,
        out_shape=jax.ShapeDtypeStruct((N, C), jnp.float32),
    )(acc2, y2g, deg_parts, b2)


def kernel(x, edge_index, W1, b1, W2, b2):
    ei = edge_index.astype(jnp.int32)
    src, dst = ei[0], ei[1]
    pad = E_PAD - E
    src_pad = jnp.concatenate([src, jnp.zeros((pad,), jnp.int32)])
    dst_pad = jnp.concatenate([dst, jnp.full((pad,), N, jnp.int32)])
    src16 = src_pad.reshape(NS, NB, EB)
    dst16 = dst_pad.reshape(NS, NB, EB)
    dst_flat = dst_pad.reshape(NW, (NB * EB) // 2)

    zeros_gw = jnp.zeros((N_ACC, GW), jnp.float32)
    x4 = x.reshape(N, 4, GW).transpose(1, 0, 2)

    deg_parts = _deg_kernel(dst_flat)
    dinv = _tc_dinv(deg_parts).reshape(N_ACC)[:N]
    dinv16 = jnp.pad(dinv.reshape(NS, SRT), ((0, 0), (0, 640 - SRT)))
    acc1 = _agg1(x4, dinv16, src16, dst16, zeros_gw)
    y2g = _tc2(acc1, x, deg_parts,
               W1, b1.reshape(1, H), W2, b2.reshape(1, C))
    acc2 = _agg2(y2g, src16, dst16, zeros_gw)
    return _tc3(acc2, y2g, deg_parts, b2.reshape(1, C))
